# baseline (speedup 1.0000x reference)
import functools

import jax
import jax.numpy as jnp
from jax import lax
from jax.experimental import pallas as pl
from jax.experimental.pallas import tpu as pltpu
from jax.experimental.pallas import tpu_sc as plsc

D = 64
NNBR = 5
NC = 2
NS = 16
NW = NC * NS
GRP = 128
CHUNK_GRPS = 2


def _sc_gather(table2, idx3):
    nw, ngrp, grp = idx3.shape
    w = table2.shape[1]
    rows_per_w = ngrp * grp
    nchunk = ngrp // CHUNK_GRPS
    chunk_rows = CHUNK_GRPS * grp
    mesh = plsc.VectorSubcoreMesh(core_axis_name="c", subcore_axis_name="s")

    @functools.partial(
        pl.kernel,
        out_type=jax.ShapeDtypeStruct((nw * rows_per_w, w), jnp.float32),
        mesh=mesh,
        scratch_types=[
            pltpu.VMEM((ngrp, grp), jnp.int32),
            pltpu.VMEM((chunk_rows, w), jnp.float32),
            pltpu.VMEM((chunk_rows, w), jnp.float32),
            pltpu.SemaphoreType.DMA,
            pltpu.SemaphoreType.DMA,
        ],
    )
    def k(table_hbm, idx_hbm, out_hbm, idx_v, rows_a, rows_b, sem_a, sem_b):
        wid = lax.axis_index("s") * NC + lax.axis_index("c")
        pltpu.sync_copy(idx_hbm.at[wid], idx_v)
        base = wid * rows_per_w
        bufs = (rows_a, rows_b)
        sems = (sem_a, sem_b)

        def fire(c):
            buf, sem = bufs[c % 2], sems[c % 2]
            return [
                pltpu.async_copy(
                    table_hbm.at[idx_v.at[c * CHUNK_GRPS + j]],
                    buf.at[pl.ds(j * grp, grp)],
                    sem,
                )
                for j in range(CHUNK_GRPS)
            ]

        pending = fire(0)
        for c in range(nchunk):
            for cp in pending:
                cp.wait()
            if c + 1 < nchunk:
                pending = fire(c + 1)
            pltpu.sync_copy(
                bufs[c % 2], out_hbm.at[pl.ds(base + c * chunk_rows, chunk_rows)]
            )

    return k(table2, idx3)


BV = 16384


def _tc_transpose(tableT):
    d, v = tableT.shape
    nbo = -(-v // (2 * BV))
    last_in = -(-v // BV) - 1

    def body(a_ref, b_ref, o_ref):
        o_ref[:, 0:d] = jnp.transpose(a_ref[...])
        o_ref[:, d : 2 * d] = jnp.transpose(b_ref[...])

    return pl.pallas_call(
        body,
        grid=(nbo,),
        in_specs=[
            pl.BlockSpec((d, BV), lambda i: (0, 2 * i)),
            pl.BlockSpec((d, BV), lambda i: (0, jnp.minimum(2 * i + 1, last_in))),
        ],
        out_specs=pl.BlockSpec((BV, 2 * d), lambda i: (i, 0)),
        out_shape=jax.ShapeDtypeStruct((nbo * BV, 2 * d), jnp.float32),
    )(tableT, tableT)


def _tc_select_posadd_t(pairs3, xt, WcT, bcT):
    b = pairs3.shape[1]
    bb = 2048

    def body(g_ref, x_ref, wc_ref, bc_ref, o_ref):
        wc = wc_ref[...]
        wc0, wc1 = wc[:, 0:1], wc[:, 1:2]
        bcc = bc_ref[...]
        for n in range(NNBR):
            gt = jnp.transpose(g_ref[n])
            w = x_ref[3 * n + 3, :].astype(jnp.int32)[None, :]
            half = ((w // BV) & 1) == 0
            sel = jnp.where(half, gt[:D, :], gt[D:, :])
            xc = x_ref[3 * n + 4, :][None, :]
            yc = x_ref[3 * n + 5, :][None, :]
            o_ref[pl.ds(n * D, D), :] = sel + xc * wc0 + yc * wc1 + bcc

    return pl.pallas_call(
        body,
        grid=(b // bb,),
        in_specs=[
            pl.BlockSpec((NNBR, bb, 2 * D), lambda j: (0, j, 0)),
            pl.BlockSpec((18, bb), lambda j: (0, j)),
            pl.BlockSpec((D, 2), lambda j: (0, 0)),
            pl.BlockSpec((D, 1), lambda j: (0, 0)),
        ],
        out_specs=pl.BlockSpec((NNBR * D, bb), lambda j: (0, j)),
        out_shape=jax.ShapeDtypeStruct((NNBR * D, b), jnp.float32),
    )(pairs3, xt, WcT, bcT)


def kernel(x, table, Wc, bc):
    b = x.shape[0]
    rows = b * NNBR
    ngrp = rows // (NW * GRP)
    xr = x.reshape(b, 6, 3)
    widx = xr[:, 1:, 0].astype(jnp.int32)
    pair_idx = (
        ((widx // (2 * BV)) * BV + (widx % BV)).T.reshape(NW, ngrp, GRP)
    )
    table2 = _tc_transpose(table.T)
    pairs = _sc_gather(table2, pair_idx)
    out = _tc_select_posadd_t(
        pairs.reshape(NNBR, b, 2 * D),
        x.T,
        Wc.T,
        bc.reshape(D, 1),
    )
    return out.reshape(NNBR, D, b).transpose(2, 0, 1)


# speedup vs baseline: 1.0091x; 1.0091x over previous
import functools

import jax
import jax.numpy as jnp
from jax import lax
from jax.experimental import pallas as pl
from jax.experimental.pallas import tpu as pltpu
from jax.experimental.pallas import tpu_sc as plsc

D = 64
NNBR = 5
NC = 2
NS = 16
NW = NC * NS
GRP = 128
CHUNK_GRPS = 4


def _sc_gather(table2, idx3):
    nw, ngrp, grp = idx3.shape
    w = table2.shape[1]
    rows_per_w = ngrp * grp
    nchunk = ngrp // CHUNK_GRPS
    chunk_rows = CHUNK_GRPS * grp
    mesh = plsc.VectorSubcoreMesh(core_axis_name="c", subcore_axis_name="s")

    @functools.partial(
        pl.kernel,
        out_type=jax.ShapeDtypeStruct((nw * rows_per_w, w), jnp.float32),
        mesh=mesh,
        scratch_types=[
            pltpu.VMEM((ngrp, grp), jnp.int32),
            pltpu.VMEM((chunk_rows, w), jnp.float32),
            pltpu.SemaphoreType.DMA,
        ],
    )
    def k(table_hbm, idx_hbm, out_hbm, idx_v, rows_v, sem):
        wid = lax.axis_index("s") * NC + lax.axis_index("c")
        pltpu.sync_copy(idx_hbm.at[wid], idx_v)
        base = wid * rows_per_w
        for c in range(nchunk):
            copies = [
                pltpu.async_copy(
                    table_hbm.at[idx_v.at[c * CHUNK_GRPS + j]],
                    rows_v.at[pl.ds(j * grp, grp)],
                    sem,
                )
                for j in range(CHUNK_GRPS)
            ]
            for cp in copies:
                cp.wait()
            pltpu.sync_copy(
                rows_v, out_hbm.at[pl.ds(base + c * chunk_rows, chunk_rows)]
            )

    return k(table2, idx3)


BV = 16384


def _tc_transpose(tableT):
    d, v = tableT.shape
    nbo = -(-v // (2 * BV))
    last_in = -(-v // BV) - 1

    def body(a_ref, b_ref, o_ref):
        o_ref[:, 0:d] = jnp.transpose(a_ref[...])
        o_ref[:, d : 2 * d] = jnp.transpose(b_ref[...])

    return pl.pallas_call(
        body,
        grid=(nbo,),
        in_specs=[
            pl.BlockSpec((d, BV), lambda i: (0, 2 * i)),
            pl.BlockSpec((d, BV), lambda i: (0, jnp.minimum(2 * i + 1, last_in))),
        ],
        out_specs=pl.BlockSpec((BV, 2 * d), lambda i: (i, 0)),
        out_shape=jax.ShapeDtypeStruct((nbo * BV, 2 * d), jnp.float32),
    )(tableT, tableT)


def _tc_select_posadd_t(pairs3, xt, WcT, bcT):
    b = pairs3.shape[1]
    bb = 2048

    def body(g_ref, x_ref, wc_ref, bc_ref, o_ref):
        wc = wc_ref[...]
        wc0, wc1 = wc[:, 0:1], wc[:, 1:2]
        bcc = bc_ref[...]
        for n in range(NNBR):
            gt = jnp.transpose(g_ref[n])
            w = x_ref[3 * n + 3, :].astype(jnp.int32)[None, :]
            half = ((w // BV) & 1) == 0
            sel = jnp.where(half, gt[:D, :], gt[D:, :])
            xc = x_ref[3 * n + 4, :][None, :]
            yc = x_ref[3 * n + 5, :][None, :]
            o_ref[pl.ds(n * D, D), :] = sel + xc * wc0 + yc * wc1 + bcc

    return pl.pallas_call(
        body,
        grid=(b // bb,),
        in_specs=[
            pl.BlockSpec((NNBR, bb, 2 * D), lambda j: (0, j, 0)),
            pl.BlockSpec((18, bb), lambda j: (0, j)),
            pl.BlockSpec((D, 2), lambda j: (0, 0)),
            pl.BlockSpec((D, 1), lambda j: (0, 0)),
        ],
        out_specs=pl.BlockSpec((NNBR * D, bb), lambda j: (0, j)),
        out_shape=jax.ShapeDtypeStruct((NNBR * D, b), jnp.float32),
    )(pairs3, xt, WcT, bcT)


def kernel(x, table, Wc, bc):
    b = x.shape[0]
    rows = b * NNBR
    ngrp = rows // (NW * GRP)
    xr = x.reshape(b, 6, 3)
    widx = xr[:, 1:, 0].astype(jnp.int32)
    pair_idx = (
        ((widx // (2 * BV)) * BV + (widx % BV)).T.reshape(NW, ngrp, GRP)
    )
    table2 = _tc_transpose(table.T)
    pairs = _sc_gather(table2, pair_idx)
    out = _tc_select_posadd_t(
        pairs.reshape(NNBR, b, 2 * D),
        x.T,
        Wc.T,
        bc.reshape(D, 1),
    )
    return out.reshape(NNBR, D, b).transpose(2, 0, 1)
